# baseline (speedup 1.0000x reference)
import functools

import jax
import jax.numpy as jnp
from jax import lax
from jax.experimental import pallas as pl
from jax.experimental.pallas import tpu as pltpu
from jax.experimental.pallas import tpu_sc as plsc

_LANES = 16
_NBUF = 4


def _sc_embed(idx_flat, tok_emb, pos_emb, t_period):
    n = idx_flat.shape[0]
    d = tok_emb.shape[1]
    nw = 32
    per_w = n // nw
    ch = t_period
    n_chunks = per_w // ch
    n_rounds = n_chunks // _NBUF
    n_grp = ch // _LANES
    d_sl = d // _LANES

    mesh = plsc.VectorSubcoreMesh(core_axis_name="c", subcore_axis_name="s")

    @functools.partial(
        pl.kernel,
        out_type=jax.ShapeDtypeStruct((n // ch, ch, d), jnp.float32),
        mesh=mesh,
        compiler_params=pltpu.CompilerParams(use_tc_tiling_on_sc=False),
        scratch_types=(
            [pltpu.VMEM((per_w,), jnp.int32),
             pltpu.VMEM((ch, d), jnp.float32)]
            + [pltpu.VMEM((ch, d), jnp.float32) for _ in range(_NBUF)]
            + [pltpu.SemaphoreType.DMA for _ in range(2 * _NBUF)]
        ),
    )
    def k(idx_hbm, tok_hbm, pos_hbm, out_hbm, idx_all, posv, *rest):
        bufs = rest[:_NBUF]
        sem_g = rest[_NBUF:2 * _NBUF]
        sem_s = rest[2 * _NBUF:]
        wid = lax.axis_index("s") * 2 + lax.axis_index("c")
        base = wid * per_w
        pltpu.sync_copy(pos_hbm.at[pl.ds(0, ch)], posv)
        pltpu.sync_copy(idx_hbm.at[pl.ds(base, per_w)], idx_all)

        def round_body(g, carry):
            gathers = []
            for b in range(_NBUF):
                u = g * _NBUF + b

                @pl.when(g > 0)
                def _drain():
                    pltpu.make_async_copy(
                        bufs[b], out_hbm.at[0], sem_s[b]).wait()

                gathers.append(pltpu.async_copy(
                    tok_hbm.at[idx_all.at[pl.ds(u * ch, ch)]],
                    bufs[b], sem_g[b]))
            for b in range(_NBUF):
                u = g * _NBUF + b
                gathers[b].wait()

                def add_rows(i, c2):
                    for r in range(_LANES):
                        j = i * _LANES + r
                        for s in range(d_sl):
                            sl = pl.ds(s * _LANES, _LANES)
                            bufs[b][j, sl] = bufs[b][j, sl] + posv[j, sl]
                    return c2

                lax.fori_loop(0, n_grp, add_rows, 0)
                for j in range(n_grp * _LANES, ch):
                    for s in range(d_sl):
                        sl = pl.ds(s * _LANES, _LANES)
                        bufs[b][j, sl] = bufs[b][j, sl] + posv[j, sl]
                pltpu.async_copy(
                    bufs[b], out_hbm.at[(base + u * ch) // ch], sem_s[b])
            return carry

        lax.fori_loop(0, n_rounds, round_body, 0)
        for b in range(_NBUF):
            pltpu.make_async_copy(
                bufs[b], out_hbm.at[0], sem_s[b]).wait()

    return k(idx_flat, tok_emb, pos_emb)


def kernel(idx, tok_emb, pos_emb):
    b, t = idx.shape
    d = tok_emb.shape[1]
    flat = idx.reshape(b * t).astype(jnp.int32)
    out = _sc_embed(flat, tok_emb, pos_emb, t)
    return out.reshape(b, t, d)
